# baseline (speedup 1.0000x reference)
import functools

import jax
import jax.numpy as jnp
from jax import lax
from jax.experimental import pallas as pl
from jax.experimental.pallas import tpu as pltpu
from jax.experimental.pallas import tpu_sc as plsc

N = 10000
D = 128
E = 320000
TAU0 = 0.5

NPAD = 10240
SLICE = 640
CHUNK = 128
EPAD = 327680
EROWS = EPAD // CHUNK
DROWS = 16
DSUP = 10
AROWS = 8
ASUP = 10
HBLK = 640


def _h_body(x_ref, w_ref, o_ref):
    i = pl.program_id(0)
    h = jnp.dot(x_ref[...], w_ref[...], preferred_element_type=jnp.float32)
    rows = lax.broadcasted_iota(jnp.int32, (HBLK, 1), 0) + i * HBLK
    o_ref[...] = jnp.where(rows < N, h, 0.0)


_h_matvec = pl.pallas_call(
    _h_body,
    grid=(NPAD // HBLK,),
    in_specs=[
        pl.BlockSpec((HBLK, D), lambda i: (i, jnp.int32(0))),
        pl.BlockSpec((D, 1), lambda i: (jnp.int32(0), jnp.int32(0))),
    ],
    out_specs=pl.BlockSpec((HBLK, 1), lambda i: (i, jnp.int32(0))),
    out_shape=jax.ShapeDtypeStruct((NPAD, 1), jnp.float32),
)


_mesh = plsc.VectorSubcoreMesh(core_axis_name="c", subcore_axis_name="s")


@functools.partial(
    pl.kernel,
    mesh=_mesh,
    out_type=[
        jax.ShapeDtypeStruct((2 * NPAD,), jnp.float32),
        jax.ShapeDtypeStruct((NPAD,), jnp.float32),
        jax.ShapeDtypeStruct((NPAD,), jnp.float32),
    ],
    compiler_params=pltpu.CompilerParams(needs_layout_passes=False),
    scratch_types=[
        pltpu.VMEM((2 * DROWS, CHUNK), jnp.int32),
        pltpu.VMEM((2 * AROWS, CHUNK), jnp.int32),
        pltpu.VMEM((2 * AROWS, CHUNK), jnp.int32),
        pltpu.VMEM((2 * AROWS, CHUNK), jnp.float32),
        pltpu.VMEM((NPAD,), jnp.float32),
        pltpu.VMEM((SLICE,), jnp.float32),
        pltpu.VMEM((SLICE,), jnp.float32),
        pltpu.VMEM((SLICE,), jnp.float32),
        pltpu.VMEM((SLICE,), jnp.float32),
        pltpu.VMEM((SLICE,), jnp.float32),
        pltpu.VMEM((SLICE,), jnp.float32),
        pltpu.VMEM((NPAD,), jnp.float32),
        pltpu.VMEM_SHARED((16 * NPAD,), jnp.float32),
        pltpu.VMEM_SHARED((NPAD,), jnp.float32),
        pltpu.SemaphoreType.DMA,
        pltpu.SemaphoreType.DMA,
        pltpu.SemaphoreType.DMA,
    ],
)
def _fused_kernel(src_hbm, dst_hbm, h_hbm,
                  acc_out, dinv_out, g_out,
                  degb, srcb, dstb, vals_v, hist, zero_v, comb, tmp,
                  hv, dinvv, gv, gall, slots, shared_acc,
                  sem0, sem1, semsc):
    c = lax.axis_index("c")
    s = lax.axis_index("s")
    w = c * jnp.int32(16) + s
    sl = s * jnp.int32(SLICE)
    ones = jnp.ones((16,), jnp.float32)

    for j in range(SLICE // 16):
        zero_v[pl.ds(j * 16, 16)] = jnp.zeros((16,), jnp.float32)

    @pl.loop(0, NPAD // SLICE)
    def _zero_hist(t):
        base = t * jnp.int32(SLICE)
        for j in range(SLICE // 16):
            hist[pl.ds(base + jnp.int32(j * 16), 16)] = jnp.zeros(
                (16,), jnp.float32)

    pltpu.sync_copy(zero_v, shared_acc.at[pl.ds(sl, SLICE)])
    pltpu.sync_copy(h_hbm.at[pl.ds(sl, SLICE)], hv)

    drow0 = s * jnp.int32(DSUP * DROWS)

    def dload(i, slot):
        return pltpu.async_copy(
            dst_hbm.at[pl.ds(drow0 + i * jnp.int32(DROWS), DROWS), :],
            degb.at[pl.ds(jnp.int32(slot * DROWS), DROWS), :],
            sem0 if slot == 0 else sem1)

    def dproc(slot):
        for j in range(DROWS):
            r = jnp.int32(slot * DROWS + j)
            for j2 in range(CHUNK // 16):
                idxv = degb[r, pl.ds(j2 * 16, 16)]
                plsc.addupdate_scatter(hist, [idxv], ones)

    dload(jnp.int32(0), 0)

    @pl.loop(0, DSUP // 2)
    def _deg_loop(io):
        i0 = io * 2
        dload(i0 + 1, 1)
        pltpu.make_async_copy(
            dst_hbm.at[pl.ds(drow0, DROWS), :],
            degb.at[pl.ds(jnp.int32(0), DROWS), :], sem0).wait()
        dproc(0)

        @pl.when(io < DSUP // 2 - 1)
        def _():
            dload(i0 + 2, 0)

        pltpu.make_async_copy(
            dst_hbm.at[pl.ds(drow0, DROWS), :],
            degb.at[pl.ds(jnp.int32(DROWS), DROWS), :], sem1).wait()
        dproc(1)

    pltpu.sync_copy(hist, slots.at[pl.ds(s * jnp.int32(NPAD), NPAD)])
    plsc.subcore_barrier()
    pltpu.sync_copy(slots.at[pl.ds(sl, SLICE)], comb)
    for t in range(1, 16):
        pltpu.sync_copy(slots.at[pl.ds(jnp.int32(t * NPAD) + sl, SLICE)], tmp)
        for j in range(SLICE // 16):
            dsl = pl.ds(j * 16, 16)
            comb[dsl] = comb[dsl] + tmp[dsl]
    plsc.subcore_barrier()

    for j in range(SLICE // 16):
        dsl = pl.ds(j * 16, 16)
        deg = comb[dsl] + 1.0
        iv = plsc.bitcast(deg, jnp.int32)
        y = plsc.bitcast(jnp.int32(0x5F3759DF) - (iv >> 1), jnp.float32)
        for _ in range(3):
            y = y * (1.5 - 0.5 * deg * y * y)
        dinvv[dsl] = y
        gv[dsl] = y * hv[dsl]
    pltpu.sync_copy(gv, slots.at[pl.ds(sl, SLICE)])

    @pl.when(c == 0)
    def _():
        pltpu.sync_copy(dinvv, dinv_out.at[pl.ds(sl, SLICE)])
        pltpu.sync_copy(gv, g_out.at[pl.ds(sl, SLICE)])

    plsc.subcore_barrier()

    arow0 = w * jnp.int32(ASUP * AROWS)

    def aload(i, slot):
        r = pl.ds(arow0 + i * jnp.int32(AROWS), AROWS)
        sb = srcb.at[pl.ds(jnp.int32(slot * AROWS), AROWS), :]
        db = dstb.at[pl.ds(jnp.int32(slot * AROWS), AROWS), :]
        sem = sem0 if slot == 0 else sem1
        pltpu.async_copy(src_hbm.at[r, :], sb, sem)
        pltpu.async_copy(dst_hbm.at[r, :], db, sem)

    def await_(slot):
        sem = sem0 if slot == 0 else sem1
        b = srcb.at[pl.ds(jnp.int32(slot * AROWS), AROWS), :]
        pltpu.make_async_copy(src_hbm.at[pl.ds(arow0, AROWS), :], b, sem).wait()
        pltpu.make_async_copy(src_hbm.at[pl.ds(arow0, AROWS), :], b, sem).wait()

    def aproc(slot):
        adds = []
        for j in range(AROWS):
            r = jnp.int32(slot * AROWS + j)
            for j2 in range(CHUNK // 16):
                idxv = srcb[r, pl.ds(j2 * 16, 16)]
                vals_v[r, pl.ds(j2 * 16, 16)] = plsc.load_gather(gall, [idxv])
            adds.append(pltpu.async_copy(
                vals_v.at[r], shared_acc.at[dstb.at[r]], semsc, add=True))
        for a in adds:
            a.wait()

    aload(jnp.int32(0), 0)
    pltpu.sync_copy(slots.at[pl.ds(jnp.int32(0), NPAD)], gall)

    @pl.loop(0, ASUP // 2)
    def _acc_loop(io):
        i0 = io * 2
        aload(i0 + 1, 1)
        await_(0)
        aproc(0)

        @pl.when(io < ASUP // 2 - 1)
        def _():
            aload(i0 + 2, 0)

        await_(1)
        aproc(1)

    plsc.subcore_barrier()
    pltpu.sync_copy(
        shared_acc.at[pl.ds(sl, SLICE)],
        acc_out.at[pl.ds(c * jnp.int32(NPAD) + sl, SLICE)],
    )


def _epi_body(a_ref, dinv_ref, g_ref, o_ref):
    acc = a_ref[0] + a_ref[1]
    o = dinv_ref[...] * (acc + g_ref[...])
    t = jnp.exp(-jnp.abs(o))
    sp = jnp.maximum(o, 0.0) + jnp.log1p(t)
    o_ref[...] = 1.0 / (sp + TAU0)


_epilogue = pl.pallas_call(
    _epi_body,
    out_shape=jax.ShapeDtypeStruct((NPAD // D, D), jnp.float32),
)


def kernel(x, edge_index, edge_attr, W):
    del edge_attr
    with jax.enable_x64(False):
        x = x.astype(jnp.float32)
        W = W.astype(jnp.float32)
        ei = edge_index.astype(jnp.int32)
        src, dst = ei[0], ei[1]
        npad_e = EPAD - E
        spread = (jnp.arange(npad_e, dtype=jnp.int32) % (NPAD - N)) + N
        src_p = jnp.concatenate([src, spread]).reshape(EROWS, CHUNK)
        dst_p = jnp.concatenate([dst, spread]).reshape(EROWS, CHUNK)

        h = _h_matvec(x, W).reshape(NPAD)
        acc, dinv, g = _fused_kernel(src_p, dst_p, h)
        temp = _epilogue(
            acc.reshape(2, NPAD // D, D),
            dinv.reshape(NPAD // D, D),
            g.reshape(NPAD // D, D),
        )
        return temp.reshape(NPAD)[:N].reshape(N, 1)


# speedup vs baseline: 1.1774x; 1.1774x over previous
import functools

import jax
import jax.numpy as jnp
from jax import lax
from jax.experimental import pallas as pl
from jax.experimental.pallas import tpu as pltpu
from jax.experimental.pallas import tpu_sc as plsc

N = 10000
D = 128
E = 320000
TAU0 = 0.5

NPAD = 10240
SLICE = 640
CHUNK = 128
SCROWS = 16
SUPER = 5
EPAD = 32 * SUPER * SCROWS * CHUNK
EROWS = EPAD // CHUNK
HBLK = 640


def _h_body(x_ref, w_ref, o_ref):
    i = pl.program_id(0)
    h = jnp.dot(x_ref[...], w_ref[...], preferred_element_type=jnp.float32)
    rows = lax.broadcasted_iota(jnp.int32, (HBLK, 1), 0) + i * HBLK
    o_ref[...] = jnp.where(rows < N, h, 0.0)


_h_matvec = pl.pallas_call(
    _h_body,
    grid=(NPAD // HBLK,),
    in_specs=[
        pl.BlockSpec((HBLK, D), lambda i: (i, jnp.int32(0))),
        pl.BlockSpec((D, 1), lambda i: (jnp.int32(0), jnp.int32(0))),
    ],
    out_specs=pl.BlockSpec((HBLK, 1), lambda i: (i, jnp.int32(0))),
    out_shape=jax.ShapeDtypeStruct((NPAD, 1), jnp.float32),
)


_mesh = plsc.VectorSubcoreMesh(core_axis_name="c", subcore_axis_name="s")


@functools.partial(
    pl.kernel,
    mesh=_mesh,
    out_type=jax.ShapeDtypeStruct((2 * NPAD,), jnp.float32),
    compiler_params=pltpu.CompilerParams(needs_layout_passes=False),
    scratch_types=[
        pltpu.VMEM((2 * SCROWS, CHUNK), jnp.int32),
        pltpu.VMEM((CHUNK,), jnp.float32),
        pltpu.VMEM((SLICE,), jnp.float32),
        pltpu.VMEM_SHARED((NPAD,), jnp.float32),
        pltpu.SemaphoreType.DMA,
        pltpu.SemaphoreType.DMA,
        pltpu.SemaphoreType.DMA,
    ],
)
def _deg_kernel(dst_hbm, deg_out, dstb, ones_v, zero_v, shared_deg,
                sem0, sem1, semsc):
    c = lax.axis_index("c")
    s = lax.axis_index("s")
    w = c * jnp.int32(16) + s
    for j in range(CHUNK // 16):
        ones_v[pl.ds(j * 16, 16)] = jnp.ones((16,), jnp.float32)
    for j in range(SLICE // 16):
        zero_v[pl.ds(j * 16, 16)] = jnp.zeros((16,), jnp.float32)
    pltpu.sync_copy(zero_v, shared_deg.at[pl.ds(s * jnp.int32(SLICE), SLICE)])
    plsc.subcore_barrier()
    row0 = w * jnp.int32(SUPER * SCROWS)
    sems = (sem0, sem1)

    def issue(i, slot):
        r = pl.ds(row0 + jnp.int32(i * SCROWS), SCROWS)
        b = dstb.at[pl.ds(jnp.int32(slot * SCROWS), SCROWS), :]
        return pltpu.async_copy(dst_hbm.at[r, :], b, sems[slot])

    pend = [issue(0, 0), None]
    for i in range(SUPER):
        slot = i & 1
        if i + 1 < SUPER:
            pend[(i + 1) & 1] = issue(i + 1, (i + 1) & 1)
        pend[slot].wait()
        adds = []
        for j in range(SCROWS):
            idx_row = dstb.at[jnp.int32(slot * SCROWS + j)]
            adds.append(pltpu.async_copy(
                ones_v, shared_deg.at[idx_row], semsc, add=True))
        for a in adds:
            a.wait()

    plsc.subcore_barrier()
    pltpu.sync_copy(
        shared_deg.at[pl.ds(s * jnp.int32(SLICE), SLICE)],
        deg_out.at[pl.ds(c * jnp.int32(NPAD) + s * jnp.int32(SLICE), SLICE)],
    )


@functools.partial(
    pl.kernel,
    mesh=_mesh,
    out_type=[
        jax.ShapeDtypeStruct((2 * NPAD,), jnp.float32),
        jax.ShapeDtypeStruct((NPAD,), jnp.float32),
        jax.ShapeDtypeStruct((NPAD,), jnp.float32),
    ],
    compiler_params=pltpu.CompilerParams(needs_layout_passes=False),
    scratch_types=[
        pltpu.VMEM((2 * SCROWS, CHUNK), jnp.int32),
        pltpu.VMEM((2 * SCROWS, CHUNK), jnp.int32),
        pltpu.VMEM((2 * SCROWS, CHUNK), jnp.float32),
        pltpu.VMEM((SLICE,), jnp.float32),
        pltpu.VMEM((SLICE,), jnp.float32),
        pltpu.VMEM((SLICE,), jnp.float32),
        pltpu.VMEM((SLICE,), jnp.float32),
        pltpu.VMEM((SLICE,), jnp.float32),
        pltpu.VMEM((NPAD,), jnp.float32),
        pltpu.VMEM_SHARED((NPAD,), jnp.float32),
        pltpu.VMEM_SHARED((NPAD,), jnp.float32),
        pltpu.SemaphoreType.DMA,
        pltpu.SemaphoreType.DMA,
        pltpu.SemaphoreType.DMA,
    ],
)
def _main_kernel(src_hbm, dst_hbm, h_hbm, deg_hbm,
                 acc_out, dinv_out, g_out,
                 srcb, dstb, vals_v, d0, gv, hv, dinvv, zero_v, gall,
                 shared_g, shared_acc, sem0, sem1, semsc):
    c = lax.axis_index("c")
    s = lax.axis_index("s")
    w = c * jnp.int32(16) + s
    sl = s * jnp.int32(SLICE)
    pltpu.sync_copy(deg_hbm.at[pl.ds(sl, SLICE)], d0)
    pltpu.sync_copy(deg_hbm.at[pl.ds(jnp.int32(NPAD) + sl, SLICE)], gv)
    pltpu.sync_copy(h_hbm.at[pl.ds(sl, SLICE)], hv)
    for j in range(SLICE // 16):
        dsl = pl.ds(j * 16, 16)
        deg = d0[dsl] + gv[dsl] + 1.0
        iv = plsc.bitcast(deg, jnp.int32)
        y = plsc.bitcast(jnp.int32(0x5F3759DF) - (iv >> 1), jnp.float32)
        for _ in range(3):
            y = y * (1.5 - 0.5 * deg * y * y)
        dinvv[dsl] = y
        gv[dsl] = y * hv[dsl]
        zero_v[dsl] = jnp.zeros((16,), jnp.float32)
    pltpu.sync_copy(gv, shared_g.at[pl.ds(sl, SLICE)])
    pltpu.sync_copy(zero_v, shared_acc.at[pl.ds(sl, SLICE)])

    @pl.when(c == 0)
    def _():
        pltpu.sync_copy(dinvv, dinv_out.at[pl.ds(sl, SLICE)])
        pltpu.sync_copy(gv, g_out.at[pl.ds(sl, SLICE)])

    plsc.subcore_barrier()
    pltpu.sync_copy(shared_g, gall)
    row0 = w * jnp.int32(SUPER * SCROWS)
    sems = (sem0, sem1)

    def issue(i, slot):
        r = pl.ds(row0 + jnp.int32(i * SCROWS), SCROWS)
        bs = srcb.at[pl.ds(jnp.int32(slot * SCROWS), SCROWS), :]
        bd = dstb.at[pl.ds(jnp.int32(slot * SCROWS), SCROWS), :]
        return (pltpu.async_copy(src_hbm.at[r, :], bs, sems[slot]),
                pltpu.async_copy(dst_hbm.at[r, :], bd, sems[slot]))

    pend = [issue(0, 0), None]
    for i in range(SUPER):
        slot = i & 1
        if i + 1 < SUPER:
            pend[(i + 1) & 1] = issue(i + 1, (i + 1) & 1)
        ca, cb = pend[slot]
        ca.wait()
        cb.wait()
        adds = []
        for j in range(SCROWS):
            r = jnp.int32(slot * SCROWS + j)
            for j2 in range(CHUNK // 16):
                idxv = srcb[r, pl.ds(j2 * 16, 16)]
                vals_v[r, pl.ds(j2 * 16, 16)] = plsc.load_gather(gall, [idxv])
            adds.append(pltpu.async_copy(
                vals_v.at[r], shared_acc.at[dstb.at[r]], semsc, add=True))
        for a in adds:
            a.wait()

    plsc.subcore_barrier()
    pltpu.sync_copy(
        shared_acc.at[pl.ds(sl, SLICE)],
        acc_out.at[pl.ds(c * jnp.int32(NPAD) + sl, SLICE)],
    )


def _epi_body(a_ref, dinv_ref, g_ref, o_ref):
    acc = a_ref[0] + a_ref[1]
    o = dinv_ref[...] * (acc + g_ref[...])
    t = jnp.exp(-jnp.abs(o))
    sp = jnp.maximum(o, 0.0) + jnp.log1p(t)
    o_ref[...] = 1.0 / (sp + TAU0)


_epilogue = pl.pallas_call(
    _epi_body,
    out_shape=jax.ShapeDtypeStruct((NPAD // D, D), jnp.float32),
)


def kernel(x, edge_index, edge_attr, W):
    del edge_attr
    x = x.astype(jnp.float32)
    W = W.astype(jnp.float32)
    ei = edge_index.astype(jnp.int32)
    src, dst = ei[0], ei[1]
    npad_e = EPAD - E
    spread = (jnp.arange(npad_e, dtype=jnp.int32) % (NPAD - N)) + N
    src_p = jnp.concatenate([src, spread]).reshape(EROWS, CHUNK)
    dst_p = jnp.concatenate([dst, spread]).reshape(EROWS, CHUNK)

    h = _h_matvec(x, W).reshape(NPAD)
    deg = _deg_kernel(dst_p)
    acc, dinv, g = _main_kernel(src_p, dst_p, h, deg)
    temp = _epilogue(
        acc.reshape(2, NPAD // D, D),
        dinv.reshape(NPAD // D, D),
        g.reshape(NPAD // D, D),
    )
    return temp.reshape(NPAD)[:N].reshape(N, 1)
